# Initial kernel scaffold; baseline (speedup 1.0000x reference)
import jax
import jax.numpy as jnp
from jax.experimental import pallas as pl


def kernel(x, edge_index, W1, b1, W2, b2):
    raise NotImplementedError("write your pallas kernel here")


# speedup vs baseline: 10.2152x; 10.2152x over previous
import functools

import jax
import jax.numpy as jnp
from jax import lax
from jax.experimental import pallas as pl
from jax.experimental.pallas import tpu as pltpu
from jax.experimental.pallas import tpu_sc as plsc

N = 10000
D = 128
E = 320000
NC = 2
NS = 16
NW = NC * NS
EPW = E // NW
CK = 80
CH = EPW // CK
NPAD = 10240
RANGE = NPAD // 2
GR = RANGE // NS
ACC_R = RANGE + NS
AZ = ACC_R // NS
DPT = NPAD // NS

_MESH = plsc.VectorSubcoreMesh(core_axis_name="c", subcore_axis_name="s")


@functools.partial(
    pl.kernel,
    mesh=_MESH,
    out_type=jax.ShapeDtypeStruct((NC, NPAD), jnp.float32),
    scratch_types=[
        pltpu.VMEM((CH, CK), jnp.int32),
        pltpu.VMEM((CK,), jnp.float32),
        pltpu.VMEM_SHARED((NPAD,), jnp.float32),
    ],
)
def _deg_kernel(dst_hbm, ones_hbm, zrow_hbm, out_hbm, dst_v, ones_v, acc):
    c = lax.axis_index("c")
    s = lax.axis_index("s")

    pltpu.sync_copy(dst_hbm.at[c, s], dst_v)
    pltpu.sync_copy(ones_hbm, ones_v)
    pltpu.sync_copy(zrow_hbm.at[0], acc.at[pl.ds(s * DPT, DPT)])
    plsc.subcore_barrier()

    def body(j, carry):
        pltpu.sync_copy(ones_v, acc.at[dst_v.at[j]], add=True)
        return carry

    lax.fori_loop(0, CH, body, 0)
    plsc.subcore_barrier()
    pltpu.sync_copy(acc.at[pl.ds(s * DPT, DPT)],
                    out_hbm.at[c, pl.ds(s * DPT, DPT)])


@functools.partial(
    pl.kernel,
    mesh=_MESH,
    out_type=jax.ShapeDtypeStruct((NC, NPAD, D), jnp.float32),
    scratch_types=[
        pltpu.VMEM((CH, CK), jnp.int32),
        pltpu.VMEM((CH, CK), jnp.int32),
        pltpu.VMEM((CH, CK), jnp.int32),
        pltpu.VMEM((CK, D), jnp.float32),
        pltpu.VMEM_SHARED((ACC_R, D), jnp.float32),
        pltpu.SemaphoreType.DMA,
    ],
)
def _msg_kernel(g_hbm, src_hbm, dst_hbm, zrows_hbm, out_hbm,
                src_v, dst_v, dst_p, buf0, acc, sem0):
    c = lax.axis_index("c")
    s = lax.axis_index("s")

    pltpu.sync_copy(src_hbm.at[c, s], src_v)
    pltpu.sync_copy(dst_hbm.at[c, s], dst_v)

    for p in range(2):
        base = p * RANGE
        garbage = RANGE + s

        def remap_row(r, carry):
            for k in range(CK // 16):
                d = dst_v[r, pl.ds(k * 16, 16)]
                m = (d >= base) & (d < base + RANGE)
                dst_p[r, pl.ds(k * 16, 16)] = jnp.where(m, d - base, garbage)
            return carry

        lax.fori_loop(0, CH, remap_row, 0)

        pltpu.sync_copy(zrows_hbm, acc.at[pl.ds(s * AZ, AZ)])
        plsc.subcore_barrier()

        def chunk(j, carry):
            pltpu.async_copy(g_hbm.at[src_v.at[j]], buf0, sem0).wait()
            pltpu.sync_copy(buf0, acc.at[dst_p.at[j]], add=True)
            return carry

        lax.fori_loop(0, CH, chunk, 0)
        plsc.subcore_barrier()
        pltpu.sync_copy(acc.at[pl.ds(s * GR, GR)],
                        out_hbm.at[c, pl.ds(base + s * GR, GR)])
        plsc.subcore_barrier()


BR = 1280
GRID = NPAD // BR


def _dinv_of(degs):
    return lax.rsqrt(degs[0] + degs[1] + 1.0)


def _mm1_body(x_ref, w_ref, degs_ref, g_ref):
    dinv = _dinv_of(degs_ref[...])
    hw = jnp.dot(x_ref[...], w_ref[...], preferred_element_type=jnp.float32)
    g_ref[...] = hw * dinv


def _mid_body(p_ref, g1_ref, degs_ref, w_ref, b_ref, g2_ref):
    dinv = _dinv_of(degs_ref[...])
    h1 = dinv * (p_ref[0] + p_ref[1] + g1_ref[...]) + b_ref[...]
    g2_ref[...] = jnp.dot(h1, w_ref[...],
                          preferred_element_type=jnp.float32) * dinv


def _out_body(q_ref, g2_ref, degs_ref, b_ref, o_ref):
    dinv = _dinv_of(degs_ref[...])
    o_ref[...] = dinv * (q_ref[0] + q_ref[1] + g2_ref[...]) + b_ref[...]


_rows = pl.BlockSpec((BR, D), lambda b: (b, 0))
_pair = pl.BlockSpec((NC, BR, D), lambda b: (0, b, 0))
_degs = pl.BlockSpec((NC, BR, 1), lambda b: (0, b, 0))
_wmat = pl.BlockSpec((D, D), lambda b: (0, 0))
_bias = pl.BlockSpec((1, D), lambda b: (0, 0))

_rows_out = jax.ShapeDtypeStruct((NPAD, D), jnp.float32)

_mm1 = pl.pallas_call(
    _mm1_body, grid=(GRID,),
    in_specs=[_rows, _wmat, _degs], out_specs=_rows, out_shape=_rows_out)

_mid = pl.pallas_call(
    _mid_body, grid=(GRID,),
    in_specs=[_pair, _rows, _degs, _wmat, _bias],
    out_specs=_rows, out_shape=_rows_out)

_out = pl.pallas_call(
    _out_body, grid=(GRID,),
    in_specs=[_pair, _rows, _degs, _bias],
    out_specs=_rows, out_shape=_rows_out)


def kernel(x, edge_index, W1, b1, W2, b2):
    srcr = edge_index[0].astype(jnp.int32).reshape(NC, NS, CH, CK)
    dstr = edge_index[1].astype(jnp.int32).reshape(NC, NS, CH, CK)
    xp = jnp.zeros((NPAD, D), jnp.float32).at[:N].set(x)
    ones = jnp.ones((CK,), jnp.float32)
    zrows = jnp.zeros((AZ, D), jnp.float32)
    zrow1 = jnp.zeros((1, DPT), jnp.float32)
    b1r = b1.reshape(1, D)
    b2r = b2.reshape(1, D)

    deg_parts = _deg_kernel(dstr, ones, zrow1)
    degs = deg_parts.reshape(NC, NPAD, 1)
    g1 = _mm1(xp, W1, degs)
    P = _msg_kernel(g1, srcr, dstr, zrows)
    g2 = _mid(P, g1, degs, W2, b1r)
    Q = _msg_kernel(g2, srcr, dstr, zrows)
    outp = _out(Q, g2, degs, b2r)
    return outp[:N]


# speedup vs baseline: 13.1982x; 1.2920x over previous
import functools

import jax
import jax.numpy as jnp
from jax import lax
from jax.experimental import pallas as pl
from jax.experimental.pallas import tpu as pltpu
from jax.experimental.pallas import tpu_sc as plsc

N = 10000
D = 128
E = 320000
NC = 2
NS = 16
NW = NC * NS
EPW = E // NW
CK = 80
CH = EPW // CK
NPAD = 10240
RANGE = NPAD // 2
GR = RANGE // NS
ACC_R = RANGE + NS
AZ = ACC_R // NS
DPT = NPAD // NS

_MESH = plsc.VectorSubcoreMesh(core_axis_name="c", subcore_axis_name="s")


@functools.partial(
    pl.kernel,
    mesh=_MESH,
    out_type=jax.ShapeDtypeStruct((NC, NPAD), jnp.float32),
    scratch_types=[
        pltpu.VMEM((CH, CK), jnp.int32),
        pltpu.VMEM((CK,), jnp.float32),
        pltpu.VMEM_SHARED((NPAD,), jnp.float32),
    ],
)
def _deg_kernel(dst_hbm, ones_hbm, zrow_hbm, out_hbm, dst_v, ones_v, acc):
    c = lax.axis_index("c")
    s = lax.axis_index("s")

    pltpu.sync_copy(dst_hbm.at[c, s], dst_v)
    pltpu.sync_copy(ones_hbm, ones_v)
    pltpu.sync_copy(zrow_hbm.at[0], acc.at[pl.ds(s * DPT, DPT)])
    plsc.subcore_barrier()

    def body(j, carry):
        pltpu.sync_copy(ones_v, acc.at[dst_v.at[j]], add=True)
        return carry

    lax.fori_loop(0, CH, body, 0)
    plsc.subcore_barrier()
    pltpu.sync_copy(acc.at[pl.ds(s * DPT, DPT)],
                    out_hbm.at[c, pl.ds(s * DPT, DPT)])


@functools.partial(
    pl.kernel,
    mesh=_MESH,
    out_type=jax.ShapeDtypeStruct((NC, NPAD, D), jnp.float32),
    scratch_types=[
        pltpu.VMEM((CH, CK), jnp.int32),
        pltpu.VMEM((CH, CK), jnp.int32),
        pltpu.VMEM((CH, CK), jnp.int32),
        pltpu.VMEM((CK, D), jnp.float32),
        pltpu.VMEM((CK, D), jnp.float32),
        pltpu.VMEM_SHARED((ACC_R, D), jnp.float32),
        pltpu.SemaphoreType.DMA,
        pltpu.SemaphoreType.DMA,
    ],
)
def _msg_kernel(g_hbm, src_hbm, dst_hbm, zrows_hbm, out_hbm,
                src_v, dst_v, dst_p, buf0, buf1, acc, sem0, sem1):
    c = lax.axis_index("c")
    s = lax.axis_index("s")

    pltpu.sync_copy(src_hbm.at[c, s], src_v)
    pltpu.sync_copy(dst_hbm.at[c, s], dst_v)

    for p in range(2):
        base = p * RANGE
        garbage = RANGE + s

        def remap_row(r, carry):
            for k in range(CK // 16):
                d = dst_v[r, pl.ds(k * 16, 16)]
                m = (d >= base) & (d < base + RANGE)
                dst_p[r, pl.ds(k * 16, 16)] = jnp.where(m, d - base, garbage)
            return carry

        lax.fori_loop(0, CH, remap_row, 0)

        pltpu.sync_copy(zrows_hbm, acc.at[pl.ds(s * AZ, AZ)])
        plsc.subcore_barrier()

        pltpu.async_copy(g_hbm.at[src_v.at[0]], buf0, sem0)

        def chunk(t, carry):
            j0 = 2 * t
            pltpu.make_async_copy(g_hbm.at[src_v.at[j0]], buf0, sem0).wait()
            pltpu.async_copy(g_hbm.at[src_v.at[j0 + 1]], buf1, sem1)
            pltpu.sync_copy(buf0, acc.at[dst_p.at[j0]], add=True)
            pltpu.make_async_copy(g_hbm.at[src_v.at[j0 + 1]], buf1, sem1).wait()
            pltpu.async_copy(g_hbm.at[src_v.at[j0 + 2]], buf0, sem0)
            pltpu.sync_copy(buf1, acc.at[dst_p.at[j0 + 1]], add=True)
            return carry

        lax.fori_loop(0, CH // 2, chunk, 0)
        pltpu.make_async_copy(g_hbm.at[src_v.at[CH - 1]], buf0, sem0).wait()
        pltpu.sync_copy(buf0, acc.at[dst_p.at[CH - 1]], add=True)
        plsc.subcore_barrier()
        pltpu.sync_copy(acc.at[pl.ds(s * GR, GR)],
                        out_hbm.at[c, pl.ds(base + s * GR, GR)])
        plsc.subcore_barrier()


BR = 1280
GRID = NPAD // BR


def _dinv_of(degs):
    return lax.rsqrt(degs[0] + degs[1] + 1.0)


def _mm1_body(x_ref, w_ref, degs_ref, g_ref):
    dinv = _dinv_of(degs_ref[...])
    hw = jnp.dot(x_ref[...], w_ref[...], preferred_element_type=jnp.float32)
    g_ref[...] = hw * dinv


def _mid_body(p_ref, g1_ref, degs_ref, w_ref, b_ref, g2_ref):
    dinv = _dinv_of(degs_ref[...])
    h1 = dinv * (p_ref[0] + p_ref[1] + g1_ref[...]) + b_ref[...]
    g2_ref[...] = jnp.dot(h1, w_ref[...],
                          preferred_element_type=jnp.float32) * dinv


def _out_body(q_ref, g2_ref, degs_ref, b_ref, o_ref):
    dinv = _dinv_of(degs_ref[...])
    o_ref[...] = dinv * (q_ref[0] + q_ref[1] + g2_ref[...]) + b_ref[...]


_rows = pl.BlockSpec((BR, D), lambda b: (b, 0))
_pair = pl.BlockSpec((NC, BR, D), lambda b: (0, b, 0))
_degs = pl.BlockSpec((NC, BR, 1), lambda b: (0, b, 0))
_wmat = pl.BlockSpec((D, D), lambda b: (0, 0))
_bias = pl.BlockSpec((1, D), lambda b: (0, 0))

_rows_out = jax.ShapeDtypeStruct((NPAD, D), jnp.float32)

_mm1 = pl.pallas_call(
    _mm1_body, grid=(GRID,),
    in_specs=[_rows, _wmat, _degs], out_specs=_rows, out_shape=_rows_out)

_mid = pl.pallas_call(
    _mid_body, grid=(GRID,),
    in_specs=[_pair, _rows, _degs, _wmat, _bias],
    out_specs=_rows, out_shape=_rows_out)

_out = pl.pallas_call(
    _out_body, grid=(GRID,),
    in_specs=[_pair, _rows, _degs, _bias],
    out_specs=_rows, out_shape=_rows_out)


def kernel(x, edge_index, W1, b1, W2, b2):
    srcr = edge_index[0].astype(jnp.int32).reshape(NC, NS, CH, CK)
    dstr = edge_index[1].astype(jnp.int32).reshape(NC, NS, CH, CK)
    xp = jnp.zeros((NPAD, D), jnp.float32).at[:N].set(x)
    ones = jnp.ones((CK,), jnp.float32)
    zrows = jnp.zeros((AZ, D), jnp.float32)
    zrow1 = jnp.zeros((1, DPT), jnp.float32)
    b1r = b1.reshape(1, D)
    b2r = b2.reshape(1, D)

    deg_parts = _deg_kernel(dstr, ones, zrow1)
    degs = deg_parts.reshape(NC, NPAD, 1)
    g1 = _mm1(xp, W1, degs)
    P = _msg_kernel(g1, srcr, dstr, zrows)
    g2 = _mid(P, g1, degs, W2, b1r)
    Q = _msg_kernel(g2, srcr, dstr, zrows)
    outp = _out(Q, g2, degs, b2r)
    return outp[:N]


# speedup vs baseline: 15.5961x; 1.1817x over previous
import functools

import jax
import jax.numpy as jnp
from jax import lax
from jax.experimental import pallas as pl
from jax.experimental.pallas import tpu as pltpu
from jax.experimental.pallas import tpu_sc as plsc

N = 10000
D = 128
E = 320000
NC = 2
NS = 16
NW = NC * NS
EPW = E // NW
CK = 125
CH = EPW // CK
NPAD = 10240
RANGE = NPAD // 2
GR = RANGE // NS
ACC_R = RANGE + NS
AZ = ACC_R // NS
DPT = NPAD // NS

_COLS = list(range(0, CK - 15, 16)) + ([CK - 16] if CK % 16 else [])

_MESH = plsc.VectorSubcoreMesh(core_axis_name="c", subcore_axis_name="s")


@functools.partial(
    pl.kernel,
    mesh=_MESH,
    out_type=jax.ShapeDtypeStruct((NC, NPAD), jnp.float32),
    scratch_types=[
        pltpu.VMEM((CH, CK), jnp.int32),
        pltpu.VMEM((CK,), jnp.float32),
        pltpu.VMEM_SHARED((NPAD,), jnp.float32),
    ],
)
def _deg_kernel(dst_hbm, ones_hbm, zrow_hbm, out_hbm, dst_v, ones_v, acc):
    c = lax.axis_index("c")
    s = lax.axis_index("s")

    pltpu.sync_copy(dst_hbm.at[c, s], dst_v)
    pltpu.sync_copy(ones_hbm, ones_v)
    pltpu.sync_copy(zrow_hbm.at[0], acc.at[pl.ds(s * DPT, DPT)])
    plsc.subcore_barrier()

    def body(j, carry):
        pltpu.sync_copy(ones_v, acc.at[dst_v.at[j]], add=True)
        return carry

    lax.fori_loop(0, CH, body, 0)
    plsc.subcore_barrier()
    pltpu.sync_copy(acc.at[pl.ds(s * DPT, DPT)],
                    out_hbm.at[c, pl.ds(s * DPT, DPT)])


@functools.partial(
    pl.kernel,
    mesh=_MESH,
    out_type=jax.ShapeDtypeStruct((NC, NPAD, D), jnp.float32),
    scratch_types=[
        pltpu.VMEM((CH, CK), jnp.int32),
        pltpu.VMEM((CH, CK), jnp.int32),
        pltpu.VMEM((CH, CK), jnp.int32),
        pltpu.VMEM((CH, CK), jnp.int32),
        pltpu.VMEM((CK, D), jnp.float32),
        pltpu.VMEM((CK, D), jnp.float32),
        pltpu.VMEM_SHARED((ACC_R, D), jnp.float32),
        pltpu.SemaphoreType.DMA,
        pltpu.SemaphoreType.DMA,
    ],
)
def _msg_kernel(g_hbm, src_hbm, dst_hbm, zrows_hbm, out_hbm,
                src_v, dst_v, dst_p0, dst_p1, buf0, buf1, acc, sem0, sem1):
    c = lax.axis_index("c")
    s = lax.axis_index("s")
    garbage = RANGE + s

    pltpu.sync_copy(src_hbm.at[c, s], src_v)
    pltpu.sync_copy(dst_hbm.at[c, s], dst_v)

    def remap_row(r, base, out_ref):
        for col in _COLS:
            d = dst_v[r, pl.ds(col, 16)]
            m = (d >= base) & (d < base + RANGE)
            out_ref[r, pl.ds(col, 16)] = jnp.where(m, d - base, garbage)

    def remap0(r, carry):
        remap_row(r, 0, dst_p0)
        return carry

    lax.fori_loop(0, CH, remap0, 0)

    for p in range(2):
        base = p * RANGE
        dst_p = dst_p0 if p == 0 else dst_p1

        pltpu.sync_copy(zrows_hbm, acc.at[pl.ds(s * AZ, AZ)])
        plsc.subcore_barrier()

        pltpu.async_copy(g_hbm.at[src_v.at[0]], buf0, sem0)

        def chunk(t, carry):
            j0 = 2 * t
            if p == 0:
                remap_row(j0, RANGE, dst_p1)
            pltpu.make_async_copy(g_hbm.at[src_v.at[j0]], buf0, sem0).wait()
            pltpu.async_copy(g_hbm.at[src_v.at[j0 + 1]], buf1, sem1)
            pltpu.sync_copy(buf0, acc.at[dst_p.at[j0]], add=True)
            if p == 0:
                remap_row(j0 + 1, RANGE, dst_p1)
            pltpu.make_async_copy(g_hbm.at[src_v.at[j0 + 1]], buf1,
                                  sem1).wait()

            @pl.when(t < CH // 2 - 1)
            def _():
                pltpu.async_copy(g_hbm.at[src_v.at[j0 + 2]], buf0, sem0)

            pltpu.sync_copy(buf1, acc.at[dst_p.at[j0 + 1]], add=True)
            return carry

        lax.fori_loop(0, CH // 2, chunk, 0)
        plsc.subcore_barrier()
        pltpu.sync_copy(acc.at[pl.ds(s * GR, GR)],
                        out_hbm.at[c, pl.ds(base + s * GR, GR)])
        plsc.subcore_barrier()


BR = 1280
GRID = NPAD // BR


def _dinv_of(degs):
    return lax.rsqrt(degs[0] + degs[1] + 1.0)


def _mm1_body(x_ref, w_ref, degs_ref, g_ref):
    dinv = _dinv_of(degs_ref[...])
    hw = jnp.dot(x_ref[...], w_ref[...], preferred_element_type=jnp.float32)
    g_ref[...] = hw * dinv


def _mid_body(p_ref, g1_ref, degs_ref, w_ref, b_ref, g2_ref):
    dinv = _dinv_of(degs_ref[...])
    h1 = dinv * (p_ref[0] + p_ref[1] + g1_ref[...]) + b_ref[...]
    g2_ref[...] = jnp.dot(h1, w_ref[...],
                          preferred_element_type=jnp.float32) * dinv


def _out_body(q_ref, g2_ref, degs_ref, b_ref, o_ref):
    dinv = _dinv_of(degs_ref[...])
    o_ref[...] = dinv * (q_ref[0] + q_ref[1] + g2_ref[...]) + b_ref[...]


_rows = pl.BlockSpec((BR, D), lambda b: (b, 0))
_pair = pl.BlockSpec((NC, BR, D), lambda b: (0, b, 0))
_degs = pl.BlockSpec((NC, BR, 1), lambda b: (0, b, 0))
_wmat = pl.BlockSpec((D, D), lambda b: (0, 0))
_bias = pl.BlockSpec((1, D), lambda b: (0, 0))

_rows_out = jax.ShapeDtypeStruct((NPAD, D), jnp.float32)

_mm1 = pl.pallas_call(
    _mm1_body, grid=(GRID,),
    in_specs=[_rows, _wmat, _degs], out_specs=_rows, out_shape=_rows_out)

_mid = pl.pallas_call(
    _mid_body, grid=(GRID,),
    in_specs=[_pair, _rows, _degs, _wmat, _bias],
    out_specs=_rows, out_shape=_rows_out)

_out = pl.pallas_call(
    _out_body, grid=(GRID,),
    in_specs=[_pair, _rows, _degs, _bias],
    out_specs=_rows, out_shape=_rows_out)


def kernel(x, edge_index, W1, b1, W2, b2):
    srcr = edge_index[0].astype(jnp.int32).reshape(NC, NS, CH, CK)
    dstr = edge_index[1].astype(jnp.int32).reshape(NC, NS, CH, CK)
    xp = jnp.zeros((NPAD, D), jnp.float32).at[:N].set(x)
    ones = jnp.ones((CK,), jnp.float32)
    zrows = jnp.zeros((AZ, D), jnp.float32)
    zrow1 = jnp.zeros((1, DPT), jnp.float32)
    b1r = b1.reshape(1, D)
    b2r = b2.reshape(1, D)

    deg_parts = _deg_kernel(dstr, ones, zrow1)
    degs = deg_parts.reshape(NC, NPAD, 1)
    g1 = _mm1(xp, W1, degs)
    P = _msg_kernel(g1, srcr, dstr, zrows)
    g2 = _mid(P, g1, degs, W2, b1r)
    Q = _msg_kernel(g2, srcr, dstr, zrows)
    outp = _out(Q, g2, degs, b2r)
    return outp[:N]


# speedup vs baseline: 18.1057x; 1.1609x over previous
import functools

import jax
import jax.numpy as jnp
from jax import lax
from jax.experimental import pallas as pl
from jax.experimental.pallas import tpu as pltpu
from jax.experimental.pallas import tpu_sc as plsc

N = 10000
D = 128
E = 320000
NC = 2
NS = 16
NW = NC * NS
EPW = E // NW
CK = 125
CH = EPW // CK
NPAD = 10240
RANGE = NPAD // 2
GR = RANGE // NS
ACC_R = RANGE + NS
AZ = ACC_R // NS
DPT = NPAD // NS

_COLS = list(range(0, CK - 15, 16)) + ([CK - 16] if CK % 16 else [])

_MESH = plsc.VectorSubcoreMesh(core_axis_name="c", subcore_axis_name="s")


@functools.partial(
    pl.kernel,
    mesh=_MESH,
    out_type=jax.ShapeDtypeStruct((NC, NPAD), jnp.float32),
    scratch_types=[
        pltpu.VMEM((CH, CK), jnp.int32),
        pltpu.VMEM((CK,), jnp.float32),
        pltpu.VMEM_SHARED((NPAD,), jnp.float32),
    ],
)
def _deg_kernel(dst_hbm, ones_hbm, zrow_hbm, out_hbm, dst_v, ones_v, acc):
    c = lax.axis_index("c")
    s = lax.axis_index("s")

    pltpu.sync_copy(dst_hbm.at[c, s], dst_v)
    pltpu.sync_copy(ones_hbm, ones_v)
    pltpu.sync_copy(zrow_hbm.at[0], acc.at[pl.ds(s * DPT, DPT)])
    plsc.subcore_barrier()

    def body(j, carry):
        pltpu.sync_copy(ones_v, acc.at[dst_v.at[j]], add=True)
        return carry

    lax.fori_loop(0, CH, body, 0)
    plsc.subcore_barrier()
    pltpu.sync_copy(acc.at[pl.ds(s * DPT, DPT)],
                    out_hbm.at[c, pl.ds(s * DPT, DPT)])


@functools.partial(
    pl.kernel,
    mesh=_MESH,
    out_type=jax.ShapeDtypeStruct((NC, NPAD, D), jnp.float32),
    scratch_types=[
        pltpu.VMEM((CH, CK), jnp.int32),
        pltpu.VMEM((CH, CK), jnp.int32),
        pltpu.VMEM((CH, CK), jnp.int32),
        pltpu.VMEM((CH, CK), jnp.int32),
        pltpu.VMEM((CK, D), jnp.float32),
        pltpu.VMEM((CK, D), jnp.float32),
        pltpu.VMEM_SHARED((ACC_R, D), jnp.float32),
        pltpu.SemaphoreType.DMA,
        pltpu.SemaphoreType.DMA,
        pltpu.SemaphoreType.DMA,
        pltpu.SemaphoreType.DMA,
    ],
)
def _msg_kernel(g_hbm, src_hbm, dst_hbm, zrows_hbm, out_hbm,
                src_v, dst_v, dst_p0, dst_p1, buf0, buf1, acc,
                sg0, sg1, ss0, ss1):
    c = lax.axis_index("c")
    s = lax.axis_index("s")
    garbage = RANGE + s
    bufs = (buf0, buf1)
    sgs = (sg0, sg1)
    sss = (ss0, ss1)
    NB = 2
    NG = CH // NB

    pltpu.sync_copy(src_hbm.at[c, s], src_v)
    pltpu.sync_copy(dst_hbm.at[c, s], dst_v)

    def remap_row(r, base, out_ref):
        for col in _COLS:
            d = dst_v[r, pl.ds(col, 16)]
            m = (d >= base) & (d < base + RANGE)
            out_ref[r, pl.ds(col, 16)] = jnp.where(m, d - base, garbage)

    def remap0(r, carry):
        remap_row(r, 0, dst_p0)
        return carry

    lax.fori_loop(0, CH, remap0, 0)

    for p in range(2):
        base = p * RANGE
        dst_p = dst_p0 if p == 0 else dst_p1

        pltpu.sync_copy(zrows_hbm, acc.at[pl.ds(s * AZ, AZ)])
        plsc.subcore_barrier()

        for jj in range(NB):
            pltpu.async_copy(g_hbm.at[src_v.at[jj]], bufs[jj], sgs[jj])

        def group(t, carry):
            j0 = NB * t
            for jj in range(NB):
                if p == 0:
                    remap_row(j0 + jj, RANGE, dst_p1)
                pltpu.make_async_copy(g_hbm.at[src_v.at[j0 + jj]],
                                      bufs[jj], sgs[jj]).wait()
                pltpu.async_copy(bufs[jj], acc.at[dst_p.at[j0 + jj]],
                                 sss[jj], add=True)
            for jj in range(NB):
                pltpu.make_async_copy(bufs[jj], acc.at[dst_p.at[j0 + jj]],
                                      sss[jj]).wait()

                @pl.when(t < NG - 1)
                def _(jj=jj, j0=j0):
                    pltpu.async_copy(g_hbm.at[src_v.at[j0 + NB + jj]],
                                     bufs[jj], sgs[jj])

            return carry

        lax.fori_loop(0, NG, group, 0)
        plsc.subcore_barrier()
        pltpu.sync_copy(acc.at[pl.ds(s * GR, GR)],
                        out_hbm.at[c, pl.ds(base + s * GR, GR)])
        plsc.subcore_barrier()


BR = 1280
GRID = NPAD // BR


def _dinv_of(degs):
    return lax.rsqrt(degs[0] + degs[1] + 1.0)


def _mm1_body(x_ref, w_ref, degs_ref, g_ref):
    dinv = _dinv_of(degs_ref[...])
    hw = jnp.dot(x_ref[...], w_ref[...], preferred_element_type=jnp.float32)
    g_ref[...] = hw * dinv


def _mid_body(p_ref, g1_ref, degs_ref, w_ref, b_ref, g2_ref):
    dinv = _dinv_of(degs_ref[...])
    h1 = dinv * (p_ref[0] + p_ref[1] + g1_ref[...]) + b_ref[...]
    g2_ref[...] = jnp.dot(h1, w_ref[...],
                          preferred_element_type=jnp.float32) * dinv


def _out_body(q_ref, g2_ref, degs_ref, b_ref, o_ref):
    dinv = _dinv_of(degs_ref[...])
    o_ref[...] = dinv * (q_ref[0] + q_ref[1] + g2_ref[...]) + b_ref[...]


_rows = pl.BlockSpec((BR, D), lambda b: (b, 0))
_pair = pl.BlockSpec((NC, BR, D), lambda b: (0, b, 0))
_degs = pl.BlockSpec((NC, BR, 1), lambda b: (0, b, 0))
_wmat = pl.BlockSpec((D, D), lambda b: (0, 0))
_bias = pl.BlockSpec((1, D), lambda b: (0, 0))

_rows_out = jax.ShapeDtypeStruct((NPAD, D), jnp.float32)

_mm1 = pl.pallas_call(
    _mm1_body, grid=(GRID,),
    in_specs=[_rows, _wmat, _degs], out_specs=_rows, out_shape=_rows_out)

_mid = pl.pallas_call(
    _mid_body, grid=(GRID,),
    in_specs=[_pair, _rows, _degs, _wmat, _bias],
    out_specs=_rows, out_shape=_rows_out)

_out = pl.pallas_call(
    _out_body, grid=(GRID,),
    in_specs=[_pair, _rows, _degs, _bias],
    out_specs=_rows, out_shape=_rows_out)


def kernel(x, edge_index, W1, b1, W2, b2):
    srcr = edge_index[0].astype(jnp.int32).reshape(NC, NS, CH, CK)
    dstr = edge_index[1].astype(jnp.int32).reshape(NC, NS, CH, CK)
    xp = jnp.zeros((NPAD, D), jnp.float32).at[:N].set(x)
    ones = jnp.ones((CK,), jnp.float32)
    zrows = jnp.zeros((AZ, D), jnp.float32)
    zrow1 = jnp.zeros((1, DPT), jnp.float32)
    b1r = b1.reshape(1, D)
    b2r = b2.reshape(1, D)

    deg_parts = _deg_kernel(dstr, ones, zrow1)
    degs = deg_parts.reshape(NC, NPAD, 1)
    g1 = _mm1(xp, W1, degs)
    P = _msg_kernel(g1, srcr, dstr, zrows)
    g2 = _mid(P, g1, degs, W2, b1r)
    Q = _msg_kernel(g2, srcr, dstr, zrows)
    outp = _out(Q, g2, degs, b2r)
    return outp[:N]
